# baseline (speedup 1.0000x reference)
import jax
import jax.numpy as jnp
from jax import lax
from jax.experimental import pallas as pl
from jax.experimental.pallas import tpu as pltpu
from jax.experimental.pallas import tpu_sc as plsc

MEM = 100000
DIM = 64
K = 8
B = 1024

BLK = 2048
MEM_PAD = 100352
NBLK = MEM_PAD // BLK

G = 128
NG = NBLK * G
GM = BLK // G
NTILE = MEM_PAD // G

NEG = -3.4e38
IMAX = 2147483647

NC = 2
NS = 16
NW = NC * NS
QPW = B // NW
NCAND = K * GM


def _extract8(s, idx):
    vals, idxs = [], []
    for _ in range(K):
        m = jnp.max(s, axis=1, keepdims=True)
        sel = jnp.min(jnp.where(s >= m, idx, IMAX), axis=1, keepdims=True)
        vals.append(m)
        idxs.append(sel)
        s = jnp.where(idx == sel, NEG, s)
    return jnp.concatenate(vals, axis=1), jnp.concatenate(idxs, axis=1)


def _sim_kernel(q_ref, k_ref, sim_ref, mg_ref, qn_s):
    step = pl.program_id(0)

    @pl.when(step == 0)
    def _init():
        qf = q_ref[...]
        qn = jnp.sqrt(jnp.sum(qf * qf, axis=1, keepdims=True))
        qn_s[...] = qf / jnp.maximum(qn, 1e-12)

    kb = k_ref[...]
    kn = jnp.sqrt(jnp.sum(kb * kb, axis=1, keepdims=True))
    kb = kb / jnp.maximum(kn, 1e-12)
    sim = lax.dot_general(qn_s[...], kb, (((1,), (1,)), ((), ())),
                          preferred_element_type=jnp.float32)
    gidx = step * BLK + lax.broadcasted_iota(jnp.int32, (B, BLK), 1)
    sim = jnp.where(gidx < MEM, sim, NEG)

    for t in range(GM):
        sim_ref[pl.ds(t * B, B), :] = sim[:, t * G:(t + 1) * G]

    mg = sim[:, 0:G]
    for k in range(1, GM):
        mg = jnp.maximum(mg, sim[:, k * G:(k + 1) * G])
    mg_ref[...] = mg


def _sim_tc(q, keys):
    return pl.pallas_call(
        _sim_kernel,
        grid=(NBLK,),
        in_specs=[
            pl.BlockSpec((B, DIM), lambda i: (0, 0)),
            pl.BlockSpec((BLK, DIM), lambda i: (i, 0)),
        ],
        out_specs=[
            pl.BlockSpec((GM * B, G), lambda i: (i, 0)),
            pl.BlockSpec((B, G), lambda i: (0, i)),
        ],
        out_shape=[
            jax.ShapeDtypeStruct((NTILE * B, G), jnp.float32),
            jax.ShapeDtypeStruct((B, NG), jnp.float32),
        ],
        scratch_shapes=[pltpu.VMEM((B, DIM), jnp.float32)],
        compiler_params=pltpu.CompilerParams(
            dimension_semantics=("arbitrary",)),
    )(q, keys)


QB = 256


def _groups_kernel(mg_ref, gid_ref):
    idx = lax.broadcasted_iota(jnp.int32, (QB, NG), 1)
    _, gi = _extract8(mg_ref[...], idx)
    gid_ref[...] = gi


def _groups_tc(mg):
    return pl.pallas_call(
        _groups_kernel,
        grid=(B // QB,),
        in_specs=[pl.BlockSpec((QB, NG), lambda i: (i, 0))],
        out_specs=pl.BlockSpec((QB, K), lambda i: (i, 0)),
        out_shape=jax.ShapeDtypeStruct((B, K), jnp.int32),
        compiler_params=pltpu.CompilerParams(
            dimension_semantics=("arbitrary",)),
    )(mg)


def _select_body(sim_hbm, gid_hbm, v_hbm, o_hbm,
                 gid_v, kidx_v, addr_v, sims_v, w_v, ti_v, rows_v, out_v, sem):
    wid = lax.axis_index("s") * NC + lax.axis_index("c")
    pltpu.sync_copy(gid_hbm.at[wid], gid_v)
    lane = lax.iota(jnp.int32, 16)

    def p1(p, _):
        g16 = gid_v[pl.ds(p * 16, 16)]
        for qq in range(2):
            qi = p * 2 + qq
            qg = wid * QPW + qi
            for j in range(K):
                gid = g16[qq * K + j]
                blk = gid // G
                g = lax.rem(gid, G)
                kidx_v[qi, pl.ds(j * 16, 16)] = blk * BLK + g + lane * G
                addr_v[qi, pl.ds(j * 16, 16)] = (
                    blk * (GM * B * G) + qg * G + g + lane * (B * G))
        return ()

    with jax.named_scope("p1_idx"):
        lax.fori_loop(0, QPW // 2, p1, (), unroll=False)

    def fire_s(qi, _):
        pltpu.async_copy(sim_hbm.at[addr_v.at[qi]], sims_v.at[qi], sem)
        return ()

    def drain_s(qi, _):
        pltpu.make_async_copy(
            sim_hbm.at[pl.ds(0, NCAND)], sims_v.at[qi], sem).wait()
        return ()

    with jax.named_scope("p2_simgather"):
        lax.fori_loop(0, QPW, fire_s, (), unroll=False)
        lax.fori_loop(0, QPW, drain_s, (), unroll=False)

    def p3(qi, _):
        sv = [sims_v[qi, pl.ds(j * 16, 16)] for j in range(K)]
        kv = [kidx_v[qi, pl.ds(j * 16, 16)] for j in range(K)]
        topv = jnp.full((16,), NEG, jnp.float32)
        topi = jnp.zeros((16,), jnp.int32)
        for t in range(K):
            vm = sv[0]
            for j in range(1, K):
                vm = jnp.maximum(vm, sv[j])
            m = jnp.max(vm)
            mf = jnp.full((16,), m, jnp.float32)
            cm = jnp.full((16,), IMAX, jnp.int32)
            for j in range(K):
                cm = jnp.minimum(cm, jnp.where(sv[j] >= mf, kv[j], IMAX))
            sel = jnp.min(cm)
            sf = jnp.full((16,), sel, jnp.int32)
            for j in range(K):
                sv[j] = jnp.where(kv[j] == sf, NEG, sv[j])
            topv = jnp.where(lane == t, mf, topv)
            topi = jnp.where(lane == t, sf, topi)
        mx = jnp.max(jnp.where(lane < K, topv, NEG))
        e = jnp.exp(topv - jnp.full((16,), mx, jnp.float32))
        e = jnp.where(lane < K, e, 0.0)
        ssum = jnp.sum(e)
        w_v[qi, :] = e / jnp.full((16,), ssum, jnp.float32)
        ti_v[qi, :] = jnp.where(lane < K, topi, 0)
        return ()

    with jax.named_scope("p3_topk"):
        lax.fori_loop(0, QPW, p3, (), unroll=False)

    def fire_v(qi, _):
        pltpu.async_copy(v_hbm.at[ti_v.at[qi]], rows_v.at[qi], sem)
        return ()

    def drain_v(qi, _):
        pltpu.make_async_copy(
            v_hbm.at[pl.ds(0, 16)], rows_v.at[qi], sem).wait()
        return ()

    with jax.named_scope("p4_valgather"):
        lax.fori_loop(0, QPW, fire_v, (), unroll=False)
        lax.fori_loop(0, QPW, drain_v, (), unroll=False)

    def p5(qi, _):
        w16 = w_v[qi, :]
        for c in range(DIM // 16):
            acc = jnp.zeros((16,), jnp.float32)
            for j in range(K):
                acc = acc + (rows_v[qi, j, pl.ds(c * 16, 16)]
                             * jnp.full((16,), w16[j], jnp.float32))
            out_v[qi, pl.ds(c * 16, 16)] = acc
        return ()

    with jax.named_scope("p5_wsum"):
        lax.fori_loop(0, QPW, p5, (), unroll=False)
    pltpu.sync_copy(out_v, o_hbm.at[pl.ds(wid * QPW, QPW)])


def _select_sc(sim2, gid8, values):
    simflat = sim2.reshape(NTILE * B * G)
    gid3 = gid8.reshape(NW, QPW * K)
    kfn = pl.kernel(
        _select_body,
        out_type=jax.ShapeDtypeStruct((B, DIM), jnp.float32),
        mesh=plsc.VectorSubcoreMesh(
            core_axis_name="c", subcore_axis_name="s",
            num_cores=NC, num_subcores=NS),
        scratch_types=[
            pltpu.VMEM((QPW * K,), jnp.int32),
            pltpu.VMEM((QPW, NCAND), jnp.int32),
            pltpu.VMEM((QPW, NCAND), jnp.int32),
            pltpu.VMEM((QPW, NCAND), jnp.float32),
            pltpu.VMEM((QPW, 16), jnp.float32),
            pltpu.VMEM((QPW, 16), jnp.int32),
            pltpu.VMEM((QPW, 16, DIM), jnp.float32),
            pltpu.VMEM((QPW, DIM), jnp.float32),
            pltpu.SemaphoreType.DMA,
        ],
        compiler_params=pltpu.CompilerParams(
            use_tc_tiling_on_sc=False, needs_layout_passes=False),
    )
    return kfn(simflat, gid3, values)


def kernel(q, keys, values):
    if q.ndim == 4:
        q = q.mean(axis=(2, 3))
    sim2, mg = _sim_tc(q, keys)
    gid8 = _groups_tc(mg)
    out = _select_sc(sim2, gid8, values)
    return out[:, :, None, None]


# speedup vs baseline: 1.4479x; 1.4479x over previous
import jax
import jax.numpy as jnp
from jax import lax
from jax.experimental import pallas as pl
from jax.experimental.pallas import tpu as pltpu
from jax.experimental.pallas import tpu_sc as plsc

MEM = 100000
DIM = 64
K = 8
B = 1024

BLK = 2048
MEM_PAD = 100352
NBLK = MEM_PAD // BLK

G = 128
NG = NBLK * G
GM = BLK // G
NTILE = MEM_PAD // G

NEG = -3.4e38
IMAX = 2147483647

NC = 2
NS = 16
NW = NC * NS
QPW = B // NW
NCAND = K * GM


def _extract8(s, idx):
    vals, idxs = [], []
    for _ in range(K):
        m = jnp.max(s, axis=1, keepdims=True)
        sel = jnp.min(jnp.where(s >= m, idx, IMAX), axis=1, keepdims=True)
        vals.append(m)
        idxs.append(sel)
        s = jnp.where(idx == sel, NEG, s)
    return jnp.concatenate(vals, axis=1), jnp.concatenate(idxs, axis=1)


def _sim_kernel(q_ref, k_ref, sim_ref, mg_ref, qn_s):
    step = pl.program_id(0)

    @pl.when(step == 0)
    def _init():
        qf = q_ref[...]
        qn = jnp.sqrt(jnp.sum(qf * qf, axis=1, keepdims=True))
        qn_s[...] = qf / jnp.maximum(qn, 1e-12)

    kb = k_ref[...]
    kn = jnp.sqrt(jnp.sum(kb * kb, axis=1, keepdims=True))
    kb = kb / jnp.maximum(kn, 1e-12)
    sim = lax.dot_general(qn_s[...], kb, (((1,), (1,)), ((), ())),
                          preferred_element_type=jnp.float32)
    gidx = step * BLK + lax.broadcasted_iota(jnp.int32, (B, BLK), 1)
    sim = jnp.where(gidx < MEM, sim, NEG)

    for t in range(GM):
        sim_ref[pl.ds(t * B, B), :] = sim[:, t * G:(t + 1) * G]

    mg = sim[:, 0:G]
    for k in range(1, GM):
        mg = jnp.maximum(mg, sim[:, k * G:(k + 1) * G])
    mg_ref[...] = mg


def _sim_tc(q, keys):
    return pl.pallas_call(
        _sim_kernel,
        grid=(NBLK,),
        in_specs=[
            pl.BlockSpec((B, DIM), lambda i: (0, 0)),
            pl.BlockSpec((BLK, DIM), lambda i: (i, 0)),
        ],
        out_specs=[
            pl.BlockSpec((GM * B, G), lambda i: (i, 0)),
            pl.BlockSpec((B, G), lambda i: (0, i)),
        ],
        out_shape=[
            jax.ShapeDtypeStruct((NTILE * B, G), jnp.float32),
            jax.ShapeDtypeStruct((B, NG), jnp.float32),
        ],
        scratch_shapes=[pltpu.VMEM((B, DIM), jnp.float32)],
        compiler_params=pltpu.CompilerParams(
            dimension_semantics=("arbitrary",)),
    )(q, keys)


QB = 256


def _groups_kernel(mg_ref, gid_ref):
    idx = lax.broadcasted_iota(jnp.int32, (QB, NG), 1)
    _, gi = _extract8(mg_ref[...], idx)
    gid_ref[...] = gi


def _groups_tc(mg):
    return pl.pallas_call(
        _groups_kernel,
        grid=(B // QB,),
        in_specs=[pl.BlockSpec((QB, NG), lambda i: (i, 0))],
        out_specs=pl.BlockSpec((QB, K), lambda i: (i, 0)),
        out_shape=jax.ShapeDtypeStruct((B, K), jnp.int32),
        compiler_params=pltpu.CompilerParams(
            dimension_semantics=("arbitrary",)),
    )(mg)


def _select_body(sim_hbm, gid_hbm, v_hbm, o_hbm,
                 gid_v, kidx_v, addr_v, sims_v, w_v, ti_v, rows_v, out_v, sem):
    wid = lax.axis_index("s") * NC + lax.axis_index("c")
    pltpu.sync_copy(gid_hbm.at[wid], gid_v)
    lane = lax.iota(jnp.int32, 16)

    def p1(p, _):
        g16 = gid_v[pl.ds(p * 16, 16)]
        for qq in range(2):
            qi = p * 2 + qq
            qg = wid * QPW + qi
            for j in range(K):
                gid = g16[qq * K + j]
                blk = gid // G
                g = lax.rem(gid, G)
                kidx_v[qi, pl.ds(j * 16, 16)] = blk * BLK + g + lane * G
                addr_v[qi, pl.ds(j * 16, 16)] = (
                    blk * (GM * B * G) + qg * G + g + lane * (B * G))
        return ()

    with jax.named_scope("p1_idx"):
        lax.fori_loop(0, QPW // 2, p1, (), unroll=False)

    def fire_s(qi, _):
        pltpu.async_copy(sim_hbm.at[addr_v.at[qi]], sims_v.at[qi], sem)
        return ()

    def drain_s(qi, _):
        pltpu.make_async_copy(
            sim_hbm.at[pl.ds(0, NCAND)], sims_v.at[qi], sem).wait()
        return ()

    with jax.named_scope("p2_simgather"):
        lax.fori_loop(0, QPW, fire_s, (), unroll=False)
        lax.fori_loop(0, QPW, drain_s, (), unroll=False)

    def p3(qi, _):
        sv = [sims_v[qi, pl.ds(j * 16, 16)] for j in range(K)]
        kv = [kidx_v[qi, pl.ds(j * 16, 16)] for j in range(K)]
        topv = jnp.full((16,), NEG, jnp.float32)
        topi = jnp.zeros((16,), jnp.int32)
        for t in range(K):
            vm = sv[0]
            for j in range(1, K):
                vm = jnp.maximum(vm, sv[j])
            m = jnp.max(vm)
            mf = jnp.full((16,), m, jnp.float32)
            cm = jnp.full((16,), IMAX, jnp.int32)
            for j in range(K):
                cm = jnp.minimum(cm, jnp.where(sv[j] >= mf, kv[j], IMAX))
            sel = jnp.min(cm)
            sf = jnp.full((16,), sel, jnp.int32)
            for j in range(K):
                sv[j] = jnp.where(kv[j] == sf, NEG, sv[j])
            topv = jnp.where(lane == t, mf, topv)
            topi = jnp.where(lane == t, sf, topi)
        mx = jnp.max(jnp.where(lane < K, topv, NEG))
        e = jnp.exp(topv - jnp.full((16,), mx, jnp.float32))
        e = jnp.where(lane < K, e, 0.0)
        ssum = jnp.sum(e)
        w_v[qi, :] = e / jnp.full((16,), ssum, jnp.float32)
        pad = 80000 + lane * 1000 + qi * 31
        ti_v[qi, :] = jnp.where(lane < K, topi, pad)
        return ()

    with jax.named_scope("p3_topk"):
        lax.fori_loop(0, QPW, p3, (), unroll=False)

    def fire_v(qi, _):
        pltpu.async_copy(v_hbm.at[ti_v.at[qi]], rows_v.at[qi], sem)
        return ()

    def drain_v(qi, _):
        pltpu.make_async_copy(
            v_hbm.at[pl.ds(0, 16)], rows_v.at[qi], sem).wait()
        return ()

    with jax.named_scope("p4_valgather"):
        lax.fori_loop(0, QPW, fire_v, (), unroll=False)
        lax.fori_loop(0, QPW, drain_v, (), unroll=False)

    def p5(qi, _):
        w16 = w_v[qi, :]
        for c in range(DIM // 16):
            acc = jnp.zeros((16,), jnp.float32)
            for j in range(K):
                acc = acc + (rows_v[qi, j, pl.ds(c * 16, 16)]
                             * jnp.full((16,), w16[j], jnp.float32))
            out_v[qi, pl.ds(c * 16, 16)] = acc
        return ()

    with jax.named_scope("p5_wsum"):
        lax.fori_loop(0, QPW, p5, (), unroll=False)
    pltpu.sync_copy(out_v, o_hbm.at[pl.ds(wid * QPW, QPW)])


def _select_sc(sim2, gid8, values):
    simflat = sim2.reshape(NTILE * B * G)
    gid3 = gid8.reshape(NW, QPW * K)
    kfn = pl.kernel(
        _select_body,
        out_type=jax.ShapeDtypeStruct((B, DIM), jnp.float32),
        mesh=plsc.VectorSubcoreMesh(
            core_axis_name="c", subcore_axis_name="s",
            num_cores=NC, num_subcores=NS),
        scratch_types=[
            pltpu.VMEM((QPW * K,), jnp.int32),
            pltpu.VMEM((QPW, NCAND), jnp.int32),
            pltpu.VMEM((QPW, NCAND), jnp.int32),
            pltpu.VMEM((QPW, NCAND), jnp.float32),
            pltpu.VMEM((QPW, 16), jnp.float32),
            pltpu.VMEM((QPW, 16), jnp.int32),
            pltpu.VMEM((QPW, 16, DIM), jnp.float32),
            pltpu.VMEM((QPW, DIM), jnp.float32),
            pltpu.SemaphoreType.DMA,
        ],
        compiler_params=pltpu.CompilerParams(
            use_tc_tiling_on_sc=False, needs_layout_passes=False),
    )
    return kfn(simflat, gid3, values)


def kernel(q, keys, values):
    if q.ndim == 4:
        q = q.mean(axis=(2, 3))
    sim2, mg = _sim_tc(q, keys)
    gid8 = _groups_tc(mg)
    out = _select_sc(sim2, gid8, values)
    return out[:, :, None, None]
